# Initial kernel scaffold; baseline (speedup 1.0000x reference)
import jax
import jax.numpy as jnp
from jax.experimental import pallas as pl


def kernel(features, edge_index):
    raise NotImplementedError("write your pallas kernel here")


# speedup vs baseline: 9.7779x; 9.7779x over previous
import functools

import jax
import jax.numpy as jnp
from jax import lax
from jax.experimental import pallas as pl
from jax.experimental.pallas import tpu as pltpu
from jax.experimental.pallas import tpu_sc as plsc

N = 10000
D = 128
E = 320000
DH = D // 2
T = 16
B = 128
C = 158
E_PAD = T * C * B
N_PAD = 10240
RPT = N_PAD // T


@functools.partial(
    pl.kernel,
    out_type=jax.ShapeDtypeStruct((2, N_PAD, DH), jnp.float32),
    mesh=plsc.VectorSubcoreMesh(core_axis_name="c", subcore_axis_name="s"),
    compiler_params=pltpu.CompilerParams(use_tc_tiling_on_sc=False),
    scratch_types=[
        pltpu.VMEM((C, B), jnp.int32),
        pltpu.VMEM((C, B), jnp.int32),
        pltpu.VMEM((2, B, DH), jnp.float32),
        pltpu.VMEM_SHARED((N_PAD, DH), jnp.float32),
        pltpu.SemaphoreType.DMA,
        pltpu.SemaphoreType.DMA,
    ],
)
def _gcn_scatter(feat_hbm, src_hbm, dst_hbm, zeros_hbm, out_hbm,
                 idx_s, idx_d, rows, acc, sem0, sem1):
    c = lax.axis_index("c")
    s = lax.axis_index("s")

    pltpu.sync_copy(src_hbm.at[c].at[s], idx_s)
    pltpu.sync_copy(dst_hbm.at[s], idx_d)
    pltpu.sync_copy(zeros_hbm, acc.at[pl.ds(s * RPT, RPT)])
    plsc.subcore_barrier()

    sems = (sem0, sem1)
    for b in range(2):
        pltpu.async_copy(feat_hbm.at[idx_s.at[b]], rows.at[b], sems[b])

    def group(g, carry):
        for b in range(2):
            j = g * 2 + b
            pltpu.make_async_copy(feat_hbm.at[idx_s.at[j]], rows.at[b],
                                  sems[b]).wait()
            pltpu.sync_copy(rows.at[b], acc.at[idx_d.at[j]], add=True)

            @pl.when(j + 2 < C)
            def _():
                pltpu.async_copy(feat_hbm.at[idx_s.at[j + 2]], rows.at[b],
                                 sems[b])
        return carry

    lax.fori_loop(0, C // 2, group, 0)

    plsc.subcore_barrier()
    pltpu.sync_copy(acc.at[pl.ds(s * RPT, RPT)],
                    out_hbm.at[c].at[pl.ds(s * RPT, RPT)])


def kernel(features, edge_index):
    src = edge_index[0]
    dst = edge_index[1]

    pad = E_PAD - E
    pad_src = jnp.arange(pad, dtype=jnp.int32) % N
    pad_dst = N + (jnp.arange(pad, dtype=jnp.int32) % (N_PAD - N))
    src_p = jnp.concatenate([src.astype(jnp.int32), pad_src])
    dst_p = jnp.concatenate([dst.astype(jnp.int32), pad_dst])

    src2 = (src_p * 2)[None, :] + jnp.arange(2, dtype=jnp.int32)[:, None]
    src3 = src2.reshape(2, T, C, B)
    dst3 = dst_p.reshape(T, C, B)
    feat2 = features.reshape(2 * N, DH)
    zeros = jnp.zeros((RPT, DH), jnp.float32)

    h2 = _gcn_scatter(feat2, src3, dst3, zeros)
    return h2.transpose(1, 0, 2).reshape(N_PAD, D)[:N]


# speedup vs baseline: 10.8671x; 1.1114x over previous
import functools

import jax
import jax.numpy as jnp
from jax import lax
from jax.experimental import pallas as pl
from jax.experimental.pallas import tpu as pltpu
from jax.experimental.pallas import tpu_sc as plsc

N = 10000
D = 128
E = 320000
DH = D // 2
T = 16
B = 128
NBUF = 4
C = 160
E_PAD = T * C * B
N_PAD = 10240
RPT = N_PAD // T


@functools.partial(
    pl.kernel,
    out_type=jax.ShapeDtypeStruct((2, N_PAD, DH), jnp.float32),
    mesh=plsc.VectorSubcoreMesh(core_axis_name="c", subcore_axis_name="s"),
    compiler_params=pltpu.CompilerParams(use_tc_tiling_on_sc=False),
    scratch_types=[
        pltpu.VMEM((C, B), jnp.int32),
        pltpu.VMEM((C, B), jnp.int32),
        pltpu.VMEM((NBUF, B, DH), jnp.float32),
        pltpu.VMEM_SHARED((N_PAD, DH), jnp.float32),
        pltpu.SemaphoreType.DMA((NBUF,)),
        pltpu.SemaphoreType.DMA((NBUF,)),
    ],
)
def _gcn_scatter(feat_hbm, src_hbm, dst_hbm, zeros_hbm, out_hbm,
                 idx_s, idx_d, rows, acc, gsem, ssem):
    c = lax.axis_index("c")
    s = lax.axis_index("s")

    pltpu.sync_copy(src_hbm.at[c].at[s], idx_s)
    pltpu.sync_copy(dst_hbm.at[s], idx_d)
    pltpu.sync_copy(zeros_hbm, acc.at[pl.ds(s * RPT, RPT)])
    plsc.subcore_barrier()

    for b in range(NBUF):
        pltpu.async_copy(feat_hbm.at[idx_s.at[b]], rows.at[b], gsem.at[b])

    def group(g, carry):
        j0 = g * NBUF
        for b in range(NBUF):
            pltpu.make_async_copy(feat_hbm.at[idx_s.at[j0 + b]], rows.at[b],
                                  gsem.at[b]).wait()
            pltpu.async_copy(rows.at[b], acc.at[idx_d.at[j0 + b]],
                             ssem.at[b], add=True)
        for b in range(NBUF):
            jn = j0 + NBUF + b

            @pl.when(jn < C)
            def _():
                pltpu.make_async_copy(rows.at[b], acc.at[idx_d.at[j0 + b]],
                                      ssem.at[b]).wait()
                pltpu.async_copy(feat_hbm.at[idx_s.at[jn]], rows.at[b],
                                 gsem.at[b])
        return carry

    lax.fori_loop(0, C // NBUF, group, 0)

    for b in range(NBUF):
        pltpu.make_async_copy(rows.at[b], acc.at[idx_d.at[C - NBUF + b]],
                              ssem.at[b]).wait()

    plsc.subcore_barrier()
    pltpu.sync_copy(acc.at[pl.ds(s * RPT, RPT)],
                    out_hbm.at[c].at[pl.ds(s * RPT, RPT)])


def kernel(features, edge_index):
    src = edge_index[0]
    dst = edge_index[1]

    pad = E_PAD - E
    pad_src = jnp.arange(pad, dtype=jnp.int32) % N
    pad_dst = N + (jnp.arange(pad, dtype=jnp.int32) % (N_PAD - N))
    src_p = jnp.concatenate([src.astype(jnp.int32), pad_src])
    dst_p = jnp.concatenate([dst.astype(jnp.int32), pad_dst])

    src2 = (src_p * 2)[None, :] + jnp.arange(2, dtype=jnp.int32)[:, None]
    src3 = src2.reshape(2, T, C, B)
    dst3 = dst_p.reshape(T, C, B)
    feat2 = features.reshape(2 * N, DH)
    zeros = jnp.zeros((RPT, DH), jnp.float32)

    h2 = _gcn_scatter(feat2, src3, dst3, zeros)
    return h2.transpose(1, 0, 2).reshape(N_PAD, D)[:N]


# speedup vs baseline: 12.3777x; 1.1390x over previous
import functools

import jax
import jax.numpy as jnp
from jax import lax
from jax.experimental import pallas as pl
from jax.experimental.pallas import tpu as pltpu
from jax.experimental.pallas import tpu_sc as plsc

N = 10000
D = 128
E = 320000
DH = D // 2
T = 16
B = 128
NBUF = 4
C = 160
E_PAD = T * C * B
N_PAD = 10240
RPT = N_PAD // T


@functools.partial(
    pl.kernel,
    out_type=jax.ShapeDtypeStruct((N_PAD, D), jnp.float32),
    mesh=plsc.VectorSubcoreMesh(core_axis_name="c", subcore_axis_name="s"),
    compiler_params=pltpu.CompilerParams(use_tc_tiling_on_sc=False),
    scratch_types=[
        pltpu.VMEM((C, B), jnp.int32),
        pltpu.VMEM((C, B), jnp.int32),
        pltpu.VMEM((NBUF, B, DH), jnp.float32),
        pltpu.VMEM_SHARED((N_PAD, DH), jnp.float32),
        pltpu.SemaphoreType.DMA((NBUF,)),
        pltpu.SemaphoreType.DMA((NBUF,)),
    ],
)
def _gcn_scatter(feat_hbm, src_hbm, dst_hbm, zeros_hbm, out_hbm,
                 idx_s, idx_d, rows, acc, gsem, ssem):
    c = lax.axis_index("c")
    s = lax.axis_index("s")

    pltpu.sync_copy(src_hbm.at[c].at[s], idx_s)
    pltpu.sync_copy(dst_hbm.at[s], idx_d)
    pltpu.sync_copy(zeros_hbm, acc.at[pl.ds(s * RPT, RPT)])
    plsc.subcore_barrier()

    for b in range(NBUF):
        pltpu.async_copy(feat_hbm.at[idx_s.at[b]], rows.at[b], gsem.at[b])

    def group(g, carry):
        j0 = g * NBUF
        for b in range(NBUF):
            pltpu.make_async_copy(feat_hbm.at[idx_s.at[j0 + b]], rows.at[b],
                                  gsem.at[b]).wait()
            pltpu.async_copy(rows.at[b], acc.at[idx_d.at[j0 + b]],
                             ssem.at[b], add=True)
        for b in range(NBUF):
            jn = j0 + NBUF + b

            @pl.when(jn < C)
            def _():
                pltpu.make_async_copy(rows.at[b], acc.at[idx_d.at[j0 + b]],
                                      ssem.at[b]).wait()
                pltpu.async_copy(feat_hbm.at[idx_s.at[jn]], rows.at[b],
                                 gsem.at[b])
        return carry

    lax.fori_loop(0, C // NBUF, group, 0)

    for b in range(NBUF):
        pltpu.make_async_copy(rows.at[b], acc.at[idx_d.at[C - NBUF + b]],
                              ssem.at[b]).wait()

    plsc.subcore_barrier()
    pltpu.sync_copy(acc.at[pl.ds(s * RPT, RPT)],
                    out_hbm.at[pl.ds(s * RPT, RPT), pl.ds(c * DH, DH)])


def kernel(features, edge_index):
    src = edge_index[0]
    dst = edge_index[1]

    pad = E_PAD - E
    pad_src = jnp.arange(pad, dtype=jnp.int32) % N
    pad_dst = N + (jnp.arange(pad, dtype=jnp.int32) % (N_PAD - N))
    src_p = jnp.concatenate([src.astype(jnp.int32), pad_src])
    dst_p = jnp.concatenate([dst.astype(jnp.int32), pad_dst])

    src2 = (src_p * 2)[None, :] + jnp.arange(2, dtype=jnp.int32)[:, None]
    src3 = src2.reshape(2, T, C, B)
    dst3 = dst_p.reshape(T, C, B)
    feat2 = features.reshape(2 * N, DH)
    zeros = jnp.zeros((RPT, DH), jnp.float32)

    h = _gcn_scatter(feat2, src3, dst3, zeros)
    return h[:N]


# speedup vs baseline: 13.0181x; 1.0517x over previous
import functools

import jax
import jax.numpy as jnp
from jax import lax
from jax.experimental import pallas as pl
from jax.experimental.pallas import tpu as pltpu
from jax.experimental.pallas import tpu_sc as plsc

N = 10000
D = 128
E = 320000
DH = D // 2
T = 16
B = 128
NBUF = 4
C = 160
E_PAD = T * C * B
N_PAD = 10240
RPT = N_PAD // T


@functools.partial(
    pl.kernel,
    out_type=jax.ShapeDtypeStruct((N, D), jnp.float32),
    mesh=plsc.VectorSubcoreMesh(core_axis_name="c", subcore_axis_name="s"),
    compiler_params=pltpu.CompilerParams(use_tc_tiling_on_sc=False),
    scratch_types=[
        pltpu.VMEM((C, B), jnp.int32),
        pltpu.VMEM((C, B), jnp.int32),
        pltpu.VMEM((NBUF, B, DH), jnp.float32),
        pltpu.VMEM_SHARED((N_PAD, DH), jnp.float32),
        pltpu.SemaphoreType.DMA((NBUF,)),
        pltpu.SemaphoreType.DMA((NBUF,)),
    ],
)
def _gcn_scatter(feat_hbm, src_hbm, dst_hbm, zeros_hbm, out_hbm,
                 idx_s, idx_d, rows, acc, gsem, ssem):
    c = lax.axis_index("c")
    s = lax.axis_index("s")

    pltpu.sync_copy(src_hbm.at[c].at[s], idx_s)
    zcp = pltpu.async_copy(zeros_hbm, acc.at[pl.ds(s * RPT, RPT)], ssem.at[0])
    dcp = pltpu.async_copy(dst_hbm.at[s], idx_d, ssem.at[1])

    for b in range(NBUF):
        pltpu.async_copy(feat_hbm.at[idx_s.at[b]], rows.at[b], gsem.at[b])

    zcp.wait()
    dcp.wait()
    plsc.subcore_barrier()

    def group(g, carry):
        j0 = g * NBUF
        for b in range(NBUF):
            pltpu.make_async_copy(feat_hbm.at[idx_s.at[j0 + b]], rows.at[b],
                                  gsem.at[b]).wait()
            pltpu.async_copy(rows.at[b], acc.at[idx_d.at[j0 + b]],
                             ssem.at[b], add=True)
        for b in range(NBUF):
            jn = j0 + NBUF + b

            @pl.when(jn < C)
            def _():
                pltpu.make_async_copy(rows.at[b], acc.at[idx_d.at[j0 + b]],
                                      ssem.at[b]).wait()
                pltpu.async_copy(feat_hbm.at[idx_s.at[jn]], rows.at[b],
                                 gsem.at[b])
        return carry

    lax.fori_loop(0, C // NBUF, group, 0)

    for b in range(NBUF):
        pltpu.make_async_copy(rows.at[b], acc.at[idx_d.at[C - NBUF + b]],
                              ssem.at[b]).wait()

    plsc.subcore_barrier()

    @pl.when(s < T - 1)
    def _():
        pltpu.sync_copy(acc.at[pl.ds(s * RPT, RPT)],
                        out_hbm.at[pl.ds(s * RPT, RPT), pl.ds(c * DH, DH)])

    @pl.when(s == T - 1)
    def _():
        last = N - (T - 1) * RPT
        pltpu.sync_copy(acc.at[pl.ds((T - 1) * RPT, last)],
                        out_hbm.at[pl.ds((T - 1) * RPT, last),
                                   pl.ds(c * DH, DH)])


def kernel(features, edge_index):
    src = edge_index[0]
    dst = edge_index[1]

    pad = E_PAD - E
    pad_src = jnp.arange(pad, dtype=jnp.int32) % N
    pad_dst = N + (jnp.arange(pad, dtype=jnp.int32) % (N_PAD - N))
    src_p = jnp.concatenate([src.astype(jnp.int32), pad_src])
    dst_p = jnp.concatenate([dst.astype(jnp.int32), pad_dst])

    src2 = (src_p * 2)[None, :] + jnp.arange(2, dtype=jnp.int32)[:, None]
    src3 = src2.reshape(2, T, C, B)
    dst3 = dst_p.reshape(T, C, B)
    feat2 = features.reshape(2 * N, DH)
    zeros = jnp.zeros((RPT, DH), jnp.float32)

    return _gcn_scatter(feat2, src3, dst3, zeros)
